# baseline (speedup 1.0000x reference)
import jax
import jax.numpy as jnp
from jax.experimental import pallas as pl
from jax.experimental.pallas import tpu as pltpu

_INPUT_DIM = 2048
_NUM_CODES = 1024
_CODE_DIM = 256
_GROUPS = 2
_T_TILE = 512

_MM_PREC = jax.lax.Precision.DEFAULT


def _vq_kernel(x_ref, w_ref, b_ref, cbt_ref, q_ref, counts_ref, psum_ref):
    step = pl.program_id(0) * pl.num_programs(1) + pl.program_id(1)

    @pl.when(step == 0)
    def _init():
        counts_ref[...] = jnp.zeros_like(counts_ref)
        psum_ref[...] = jnp.zeros_like(psum_ref)

    x = x_ref[0]
    logits = jax.lax.dot_general(
        w_ref[...], x, (((1,), (0,)), ((), ())),
        precision=_MM_PREC, preferred_element_type=jnp.float32)
    logits = logits + b_ref[...]
    l3 = logits.reshape(_GROUPS, _NUM_CODES, _T_TILE)

    m = jnp.max(l3, axis=1)
    iota = jax.lax.broadcasted_iota(jnp.int32, l3.shape, 1)
    k = jnp.min(jnp.where(l3 == m[:, None, :], iota, _NUM_CODES), axis=1)
    onehot = (iota == k[:, None, :]).astype(jnp.float32)

    ones_t = jnp.ones((_T_TILE, 1), jnp.float32)
    counts_ref[...] += jax.lax.dot_general(
        onehot.reshape(_GROUPS * _NUM_CODES, _T_TILE), ones_t,
        (((1,), (0,)), ((), ())),
        precision=_MM_PREC, preferred_element_type=jnp.float32)

    p = jnp.exp(l3 - m[:, None, :])
    rinv = 1.0 / jnp.sum(p, axis=1)

    for g in range(_GROUPS):
        psum_ref[g * _NUM_CODES:(g + 1) * _NUM_CODES, :] += jax.lax.dot_general(
            p[g], rinv[g].reshape(_T_TILE, 1), (((1,), (0,)), ((), ())),
            precision=jax.lax.Precision.HIGHEST,
            preferred_element_type=jnp.float32)
        qg = jax.lax.dot_general(
            cbt_ref[g], onehot[g], (((1,), (0,)), ((), ())),
            precision=_MM_PREC, preferred_element_type=jnp.float32)
        q_ref[0, g * _CODE_DIM:(g + 1) * _CODE_DIM, :] = qg


def kernel(x, W, b, codebook):
    bsz, fsz, tsz = x.shape
    gv = _GROUPS * _NUM_CODES
    n_tok = bsz * tsz
    cbt = jnp.transpose(codebook[0], (0, 2, 1))
    b2 = b.reshape(gv, 1)

    grid = (bsz, tsz // _T_TILE)
    q, counts, psum = pl.pallas_call(
        _vq_kernel,
        grid=grid,
        in_specs=[
            pl.BlockSpec((1, fsz, _T_TILE), lambda i, t: (i, 0, t)),
            pl.BlockSpec((gv, fsz), lambda i, t: (0, 0)),
            pl.BlockSpec((gv, 1), lambda i, t: (0, 0)),
            pl.BlockSpec((_GROUPS, _CODE_DIM, _NUM_CODES), lambda i, t: (0, 0, 0)),
        ],
        out_specs=[
            pl.BlockSpec((1, _GROUPS * _CODE_DIM, _T_TILE), lambda i, t: (i, 0, t)),
            pl.BlockSpec((gv, 1), lambda i, t: (0, 0)),
            pl.BlockSpec((gv, 1), lambda i, t: (0, 0)),
        ],
        out_shape=[
            jax.ShapeDtypeStruct((bsz, _GROUPS * _CODE_DIM, tsz), jnp.float32),
            jax.ShapeDtypeStruct((gv, 1), jnp.float32),
            jax.ShapeDtypeStruct((gv, 1), jnp.float32),
        ],
        compiler_params=pltpu.CompilerParams(
            dimension_semantics=("arbitrary", "arbitrary"),
        ),
    )(x, W, b2, cbt)

    hard_probs = counts.reshape(_GROUPS, _NUM_CODES) / n_tok
    code_perplexity = jnp.sum(
        jnp.exp(-jnp.sum(hard_probs * jnp.log(hard_probs + 1e-7), axis=-1)))
    avg_probs = psum.reshape(_GROUPS, _NUM_CODES) / n_tok
    prob_perplexity = jnp.sum(
        jnp.exp(-jnp.sum(avg_probs * jnp.log(avg_probs + 1e-7), axis=-1)))
    num_vars = _NUM_CODES * _GROUPS
    diversity = (num_vars - prob_perplexity) / num_vars
    return q, diversity, code_perplexity, prob_perplexity


# speedup vs baseline: 1.4683x; 1.4683x over previous
import jax
import jax.numpy as jnp
from jax.experimental import pallas as pl
from jax.experimental.pallas import tpu as pltpu

_INPUT_DIM = 2048
_NUM_CODES = 1024
_CODE_DIM = 256
_GROUPS = 2
_T_TILE = 1024

_MM_PREC = jax.lax.Precision.DEFAULT


def _vq_kernel(x_ref, w_ref, b_ref, cbt_ref, q_ref, counts_ref, psum_ref):
    step = pl.program_id(0) * pl.num_programs(1) + pl.program_id(1)

    @pl.when(step == 0)
    def _init():
        counts_ref[...] = jnp.zeros_like(counts_ref)
        psum_ref[...] = jnp.zeros_like(psum_ref)

    x = x_ref[0]
    logits = jax.lax.dot_general(
        w_ref[...], x, (((1,), (0,)), ((), ())),
        precision=_MM_PREC, preferred_element_type=jnp.float32)
    logits = logits + b_ref[...]
    l3 = logits.reshape(_GROUPS, _NUM_CODES, _T_TILE)

    m = jnp.max(l3, axis=1)
    iota = jax.lax.broadcasted_iota(jnp.int32, l3.shape, 1)
    k = jnp.min(jnp.where(l3 == m[:, None, :], iota, _NUM_CODES), axis=1)
    onehot = (iota == k[:, None, :]).astype(jnp.float32)

    counts_ref[...] += jnp.sum(onehot, axis=2)

    p = jnp.exp(l3 - m[:, None, :])
    rinv = 1.0 / jnp.sum(p, axis=1)
    psum_ref[...] += jnp.sum(p * rinv[:, None, :], axis=2)

    for g in range(_GROUPS):
        qg = jax.lax.dot_general(
            cbt_ref[g], onehot[g], (((1,), (0,)), ((), ())),
            precision=_MM_PREC, preferred_element_type=jnp.float32)
        q_ref[0, g * _CODE_DIM:(g + 1) * _CODE_DIM, :] = qg


def kernel(x, W, b, codebook):
    bsz, fsz, tsz = x.shape
    gv = _GROUPS * _NUM_CODES
    n_tok = bsz * tsz
    cbt = jnp.transpose(codebook[0], (0, 2, 1))
    b2 = b.reshape(gv, 1)

    grid = (bsz, tsz // _T_TILE)
    q, counts, psum = pl.pallas_call(
        _vq_kernel,
        grid=grid,
        in_specs=[
            pl.BlockSpec((1, fsz, _T_TILE), lambda i, t: (i, 0, t)),
            pl.BlockSpec((gv, fsz), lambda i, t: (0, 0)),
            pl.BlockSpec((gv, 1), lambda i, t: (0, 0)),
            pl.BlockSpec((_GROUPS, _CODE_DIM, _NUM_CODES), lambda i, t: (0, 0, 0)),
        ],
        out_specs=[
            pl.BlockSpec((1, _GROUPS * _CODE_DIM, _T_TILE), lambda i, t: (i, 0, t)),
            pl.BlockSpec((_GROUPS, _NUM_CODES), lambda i, t: (0, 0)),
            pl.BlockSpec((_GROUPS, _NUM_CODES), lambda i, t: (0, 0)),
        ],
        out_shape=[
            jax.ShapeDtypeStruct((bsz, _GROUPS * _CODE_DIM, tsz), jnp.float32),
            jax.ShapeDtypeStruct((_GROUPS, _NUM_CODES), jnp.float32),
            jax.ShapeDtypeStruct((_GROUPS, _NUM_CODES), jnp.float32),
        ],
        compiler_params=pltpu.CompilerParams(
            dimension_semantics=("arbitrary", "arbitrary"),
        ),
    )(x, W, b2, cbt)

    hard_probs = counts / n_tok
    code_perplexity = jnp.sum(
        jnp.exp(-jnp.sum(hard_probs * jnp.log(hard_probs + 1e-7), axis=-1)))
    avg_probs = psum / n_tok
    prob_perplexity = jnp.sum(
        jnp.exp(-jnp.sum(avg_probs * jnp.log(avg_probs + 1e-7), axis=-1)))
    num_vars = _NUM_CODES * _GROUPS
    diversity = (num_vars - prob_perplexity) / num_vars
    return q, diversity, code_perplexity, prob_perplexity


# speedup vs baseline: 1.5094x; 1.0280x over previous
import jax
import jax.numpy as jnp
from jax.experimental import pallas as pl
from jax.experimental.pallas import tpu as pltpu

_INPUT_DIM = 2048
_NUM_CODES = 1024
_CODE_DIM = 256
_GROUPS = 2
_T_TILE = 1024

_MM_PREC = jax.lax.Precision.DEFAULT


def _vq_kernel(x_ref, w_ref, b_ref, cbt_ref, q_ref, div_ref, cp_ref, pp_ref,
               counts_ref, psum_ref):
    step = pl.program_id(0) * pl.num_programs(1) + pl.program_id(1)
    nsteps = pl.num_programs(0) * pl.num_programs(1)

    @pl.when(step == 0)
    def _init():
        counts_ref[...] = jnp.zeros_like(counts_ref)
        psum_ref[...] = jnp.zeros_like(psum_ref)

    x = x_ref[0]
    logits = jax.lax.dot_general(
        w_ref[...], x, (((1,), (0,)), ((), ())),
        precision=_MM_PREC, preferred_element_type=jnp.float32)
    logits = logits + b_ref[...]
    l3 = logits.reshape(_GROUPS, _NUM_CODES, _T_TILE)

    m = jnp.max(l3, axis=1)
    iota = jax.lax.broadcasted_iota(jnp.int32, l3.shape, 1)
    k = jnp.min(jnp.where(l3 == m[:, None, :], iota, _NUM_CODES), axis=1)
    onehot = (iota == k[:, None, :]).astype(jnp.float32)

    counts_ref[...] += jnp.sum(onehot, axis=2)

    p = jnp.exp(l3 - m[:, None, :])
    rinv = 1.0 / jnp.sum(p, axis=1)
    psum_ref[...] += jnp.sum(p * rinv[:, None, :], axis=2)

    for g in range(_GROUPS):
        qg = jax.lax.dot_general(
            cbt_ref[g], onehot[g], (((1,), (0,)), ((), ())),
            precision=_MM_PREC, preferred_element_type=jnp.float32)
        q_ref[0, g * _CODE_DIM:(g + 1) * _CODE_DIM, :] = qg

    @pl.when(step == nsteps - 1)
    def _scalars():
        n_tok = jnp.float32(pl.num_programs(0) * pl.num_programs(1) * _T_TILE)
        hard_probs = counts_ref[...] / n_tok
        cp = jnp.sum(jnp.exp(
            -jnp.sum(hard_probs * jnp.log(hard_probs + 1e-7), axis=-1)))
        avg_probs = psum_ref[...] / n_tok
        pp = jnp.sum(jnp.exp(
            -jnp.sum(avg_probs * jnp.log(avg_probs + 1e-7), axis=-1)))
        num_vars = jnp.float32(_NUM_CODES * _GROUPS)
        cp_ref[...] = cp.reshape(1, 1)
        pp_ref[...] = pp.reshape(1, 1)
        div_ref[...] = ((num_vars - pp) / num_vars).reshape(1, 1)


def kernel(x, W, b, codebook):
    bsz, fsz, tsz = x.shape
    gv = _GROUPS * _NUM_CODES
    n_tok = bsz * tsz
    cbt = jnp.transpose(codebook[0], (0, 2, 1))
    b2 = b.reshape(gv, 1)

    grid = (bsz, tsz // _T_TILE)
    q, div, cp, pp, _, _ = pl.pallas_call(
        _vq_kernel,
        grid=grid,
        in_specs=[
            pl.BlockSpec((1, fsz, _T_TILE), lambda i, t: (i, 0, t)),
            pl.BlockSpec((gv, fsz), lambda i, t: (0, 0)),
            pl.BlockSpec((gv, 1), lambda i, t: (0, 0)),
            pl.BlockSpec((_GROUPS, _CODE_DIM, _NUM_CODES), lambda i, t: (0, 0, 0)),
        ],
        out_specs=[
            pl.BlockSpec((1, _GROUPS * _CODE_DIM, _T_TILE), lambda i, t: (i, 0, t)),
            pl.BlockSpec((1, 1), lambda i, t: (0, 0)),
            pl.BlockSpec((1, 1), lambda i, t: (0, 0)),
            pl.BlockSpec((1, 1), lambda i, t: (0, 0)),
            pl.BlockSpec((_GROUPS, _NUM_CODES), lambda i, t: (0, 0)),
            pl.BlockSpec((_GROUPS, _NUM_CODES), lambda i, t: (0, 0)),
        ],
        out_shape=[
            jax.ShapeDtypeStruct((bsz, _GROUPS * _CODE_DIM, tsz), jnp.float32),
            jax.ShapeDtypeStruct((1, 1), jnp.float32),
            jax.ShapeDtypeStruct((1, 1), jnp.float32),
            jax.ShapeDtypeStruct((1, 1), jnp.float32),
            jax.ShapeDtypeStruct((_GROUPS, _NUM_CODES), jnp.float32),
            jax.ShapeDtypeStruct((_GROUPS, _NUM_CODES), jnp.float32),
        ],
        compiler_params=pltpu.CompilerParams(
            dimension_semantics=("arbitrary", "arbitrary"),
        ),
    )(x, W, b2, cbt)

    return (q, div.reshape(()), cp.reshape(()), pp.reshape(()))
